# Initial kernel scaffold; baseline (speedup 1.0000x reference)
import functools

import jax
import jax.numpy as jnp
from jax import lax
from jax.experimental import pallas as pl
from jax.experimental.pallas import tpu as pltpu
from jax.experimental.pallas import tpu_sc as plsc

NN = 10000
NP = 10240
EE = 320000
NC = 2
NS = 16
CH = 128
KROW = 8
TILE_E = 10240
EP = TILE_E * NC * NS
OUTER = TILE_E // (KROW * CH)
ROWS_PER_TILE = NP // NS

_mesh = plsc.VectorSubcoreMesh(core_axis_name="c", subcore_axis_name="s")


def _degree_kernel(src2, dst2, zeros16, ones16):
    @functools.partial(
        pl.kernel,
        mesh=_mesh,
        out_type=jax.ShapeDtypeStruct((NC, 2, NP, 16), jnp.float32),
        scratch_types=[
            pltpu.VMEM((KROW, CH), jnp.int32),
            pltpu.VMEM((KROW, CH), jnp.int32),
            pltpu.VMEM((CH, 16), jnp.float32),
            pltpu.VMEM_SHARED((NP, 16), jnp.float32),
            pltpu.VMEM_SHARED((NP, 16), jnp.float32),
        ],
    )
    def deg(src_hbm, dst_hbm, zeros_hbm, ones_hbm, out_hbm,
            sidx, didx, ones_v, acc_s, acc_d):
        cid = lax.axis_index("c")
        sid = lax.axis_index("s")
        wid = sid * NC + cid
        pltpu.sync_copy(ones_hbm, ones_v)
        sl = pl.ds(sid * ROWS_PER_TILE, ROWS_PER_TILE)
        pltpu.sync_copy(zeros_hbm.at[sl], acc_s.at[sl])
        pltpu.sync_copy(zeros_hbm.at[sl], acc_d.at[sl])
        plsc.subcore_barrier()
        row0 = wid * (TILE_E // CH)

        @pl.loop(0, OUTER)
        def _(i):
            base = row0 + i * KROW
            pltpu.sync_copy(src_hbm.at[pl.ds(base, KROW)], sidx)
            pltpu.sync_copy(dst_hbm.at[pl.ds(base, KROW)], didx)
            for j in range(KROW):
                pltpu.sync_copy(ones_v, acc_s.at[sidx.at[j]], add=True)
                pltpu.sync_copy(ones_v, acc_d.at[didx.at[j]], add=True)

        plsc.subcore_barrier()
        pltpu.sync_copy(acc_s.at[sl], out_hbm.at[cid, 0].at[sl])
        pltpu.sync_copy(acc_d.at[sl], out_hbm.at[cid, 1].at[sl])

    return deg(src2, dst2, zeros16, ones16)


def _propagate(table, src2, dst2, zeros, F):
    @functools.partial(
        pl.kernel,
        mesh=_mesh,
        out_type=jax.ShapeDtypeStruct((NC, NP, F), jnp.float32),
        scratch_types=[
            pltpu.VMEM((KROW, CH), jnp.int32),
            pltpu.VMEM((KROW, CH), jnp.int32),
            pltpu.VMEM((CH, F), jnp.float32),
            pltpu.VMEM_SHARED((NP, F), jnp.float32),
            pltpu.SemaphoreType.DMA,
        ],
    )
    def prop(table_hbm, src_hbm, dst_hbm, zeros_hbm, out_hbm,
             sidx, didx, rows, acc, sem):
        cid = lax.axis_index("c")
        sid = lax.axis_index("s")
        wid = sid * NC + cid
        sl = pl.ds(sid * ROWS_PER_TILE, ROWS_PER_TILE)
        pltpu.sync_copy(zeros_hbm.at[sl], acc.at[sl])
        plsc.subcore_barrier()
        row0 = wid * (TILE_E // CH)

        @pl.loop(0, OUTER)
        def _(i):
            base = row0 + i * KROW
            pltpu.sync_copy(src_hbm.at[pl.ds(base, KROW)], sidx)
            pltpu.sync_copy(dst_hbm.at[pl.ds(base, KROW)], didx)
            for j in range(KROW):
                pltpu.async_copy(table_hbm.at[sidx.at[j]], rows, sem).wait()
                pltpu.sync_copy(rows, acc.at[didx.at[j]], add=True)

        plsc.subcore_barrier()
        pltpu.sync_copy(acc.at[sl], out_hbm.at[cid].at[sl])

    return prop(table, src2, dst2, zeros)


def _stage_a(degs, feats, w_enc):
    def body(degs_ref, x_ref, w_ref, oisq_ref, iisq_ref, y1_ref):
        dout = degs_ref[0, 0, :, 0:1] + degs_ref[1, 0, :, 0:1]
        din = degs_ref[0, 1, :, 0:1] + degs_ref[1, 1, :, 0:1]
        oisq = lax.rsqrt(jnp.maximum(dout, 1.0))
        iisq = lax.rsqrt(jnp.maximum(din, 1.0))
        oisq_ref[...] = oisq
        iisq_ref[...] = iisq
        y1_ref[...] = jnp.dot(x_ref[...] * oisq, w_ref[...],
                              preferred_element_type=jnp.float32)

    return pl.pallas_call(
        body,
        out_shape=(
            jax.ShapeDtypeStruct((NP, 1), jnp.float32),
            jax.ShapeDtypeStruct((NP, 1), jnp.float32),
            jax.ShapeDtypeStruct((NP, 64), jnp.float32),
        ),
    )(degs, feats, w_enc)


def _stage_b(parts, iisq, oisq, b_enc, w_emb):
    def body(p_ref, ii_ref, oi_ref, b_ref, w_ref, y2_ref):
        agg = p_ref[0] + p_ref[1]
        h = jnp.maximum(agg * ii_ref[...] + b_ref[...], 0.0)
        y2_ref[...] = jnp.dot(h * oi_ref[...], w_ref[...],
                              preferred_element_type=jnp.float32)

    return pl.pallas_call(
        body,
        out_shape=jax.ShapeDtypeStruct((NP, 32), jnp.float32),
    )(parts, iisq, oisq, b_enc, w_emb)


def _stage_c(parts, iisq, oisq, b_emb):
    def body(p_ref, ii_ref, oi_ref, b_ref, z_ref, y3_ref):
        z = (p_ref[0] + p_ref[1]) * ii_ref[...] + b_ref[...]
        z_ref[...] = z
        y3_ref[...] = z * oi_ref[...]

    return pl.pallas_call(
        body,
        out_shape=(
            jax.ShapeDtypeStruct((NP, 32), jnp.float32),
            jax.ShapeDtypeStruct((NP, 32), jnp.float32),
        ),
    )(parts, iisq, oisq, b_emb)


def _stage_d(parts, iisq, b_dec, w_dec):
    def body(p_ref, ii_ref, b_ref, w_ref, rec_ref):
        agg = (p_ref[0] + p_ref[1]) * ii_ref[...]
        rec_ref[...] = jnp.dot(agg, w_ref[...],
                               preferred_element_type=jnp.float32) + b_ref[...]

    return pl.pallas_call(
        body,
        out_shape=jax.ShapeDtypeStruct((NP, 128), jnp.float32),
    )(parts, iisq, b_dec, w_dec)


def kernel(features, edge_index, W_enc, b_enc, W_emb, b_emb, W_dec, b_dec):
    src = edge_index[0]
    dst = edge_index[1]
    pad = jnp.full((EP - EE,), NN, dtype=jnp.int32)
    src2 = jnp.concatenate([src, pad]).reshape(EP // CH, CH)
    dst2 = jnp.concatenate([dst, pad]).reshape(EP // CH, CH)

    feats = jnp.pad(features, ((0, NP - NN), (0, 0)))
    zeros16 = jnp.zeros((NP, 16), jnp.float32)
    ones16 = jnp.ones((CH, 16), jnp.float32)
    zeros64 = jnp.zeros((NP, 64), jnp.float32)
    zeros32 = jnp.zeros((NP, 32), jnp.float32)

    degs = _degree_kernel(src2, dst2, zeros16, ones16)
    oisq, iisq, y1 = _stage_a(degs, feats, W_enc)
    agg1 = _propagate(y1, src2, dst2, zeros64, 64)
    y2 = _stage_b(agg1, iisq, oisq, b_enc.reshape(1, -1), W_emb)
    agg2 = _propagate(y2, src2, dst2, zeros32, 32)
    z_full, y3 = _stage_c(agg2, iisq, oisq, b_emb.reshape(1, -1))
    agg3 = _propagate(y3, src2, dst2, zeros32, 32)
    recon_full = _stage_d(agg3, iisq, b_dec.reshape(1, -1), W_dec)

    return (z_full[:NN], recon_full[:NN])


# speedup vs baseline: 5.2670x; 5.2670x over previous
import functools

import jax
import jax.numpy as jnp
from jax import lax
from jax.experimental import pallas as pl
from jax.experimental.pallas import tpu as pltpu
from jax.experimental.pallas import tpu_sc as plsc

NN = 10000
NP = 10240
EE = 320000
NC = 2
NS = 16
CH = 128
KROW = 8
TILE_E = 10240
EP = TILE_E * NC * NS
OUTER = TILE_E // (KROW * CH)
ROWS_PER_TILE = NP // NS

_mesh = plsc.VectorSubcoreMesh(core_axis_name="c", subcore_axis_name="s")
_sc_params = pltpu.CompilerParams(use_tc_tiling_on_sc=False)


def _degree_kernel(src2, dst2, zeros16, ones16):
    @functools.partial(
        pl.kernel,
        mesh=_mesh,
        out_type=jax.ShapeDtypeStruct((NC, 2, NP, 16), jnp.float32),
        compiler_params=_sc_params,
        scratch_types=[
            pltpu.VMEM((KROW, CH), jnp.int32),
            pltpu.VMEM((KROW, CH), jnp.int32),
            pltpu.VMEM((CH, 16), jnp.float32),
            pltpu.VMEM_SHARED((NP, 16), jnp.float32),
            pltpu.VMEM_SHARED((NP, 16), jnp.float32),
        ],
    )
    def deg(src_hbm, dst_hbm, zeros_hbm, ones_hbm, out_hbm,
            sidx, didx, ones_v, acc_s, acc_d):
        cid = lax.axis_index("c")
        sid = lax.axis_index("s")
        wid = sid * NC + cid
        pltpu.sync_copy(ones_hbm, ones_v)
        sl = pl.ds(sid * ROWS_PER_TILE, ROWS_PER_TILE)
        pltpu.sync_copy(zeros_hbm.at[sl], acc_s.at[sl])
        pltpu.sync_copy(zeros_hbm.at[sl], acc_d.at[sl])
        plsc.subcore_barrier()
        row0 = wid * (TILE_E // CH)

        @pl.loop(0, OUTER)
        def _(i):
            base = row0 + i * KROW
            pltpu.sync_copy(src_hbm.at[pl.ds(base, KROW)], sidx)
            pltpu.sync_copy(dst_hbm.at[pl.ds(base, KROW)], didx)
            for j in range(KROW):
                pltpu.sync_copy(ones_v, acc_s.at[sidx.at[j]], add=True)
                pltpu.sync_copy(ones_v, acc_d.at[didx.at[j]], add=True)

        plsc.subcore_barrier()
        pltpu.sync_copy(acc_s.at[sl], out_hbm.at[cid, 0].at[sl])
        pltpu.sync_copy(acc_d.at[sl], out_hbm.at[cid, 1].at[sl])

    return deg(src2, dst2, zeros16, ones16)


def _propagate(table, src2, dst2, zeros, F):
    @functools.partial(
        pl.kernel,
        mesh=_mesh,
        out_type=jax.ShapeDtypeStruct((NC, NP, F), jnp.float32),
        compiler_params=_sc_params,
        scratch_types=[
            pltpu.VMEM((KROW, CH), jnp.int32),
            pltpu.VMEM((KROW, CH), jnp.int32),
            pltpu.VMEM((CH, F), jnp.float32),
            pltpu.VMEM_SHARED((NP, F), jnp.float32),
            pltpu.SemaphoreType.DMA,
        ],
    )
    def prop(table_hbm, src_hbm, dst_hbm, zeros_hbm, out_hbm,
             sidx, didx, rows, acc, sem):
        cid = lax.axis_index("c")
        sid = lax.axis_index("s")
        wid = sid * NC + cid
        sl = pl.ds(sid * ROWS_PER_TILE, ROWS_PER_TILE)
        pltpu.sync_copy(zeros_hbm.at[sl], acc.at[sl])
        plsc.subcore_barrier()
        row0 = wid * (TILE_E // CH)

        @pl.loop(0, OUTER)
        def _(i):
            base = row0 + i * KROW
            pltpu.sync_copy(src_hbm.at[pl.ds(base, KROW)], sidx)
            pltpu.sync_copy(dst_hbm.at[pl.ds(base, KROW)], didx)
            for j in range(KROW):
                pltpu.async_copy(table_hbm.at[sidx.at[j]], rows, sem).wait()
                pltpu.sync_copy(rows, acc.at[didx.at[j]], add=True)

        plsc.subcore_barrier()
        pltpu.sync_copy(acc.at[sl], out_hbm.at[cid].at[sl])

    return prop(table, src2, dst2, zeros)


def _stage_a(degs, feats, w_enc):
    def body(degs_ref, x_ref, w_ref, oisq_ref, iisq_ref, y1_ref):
        dout = degs_ref[0, 0, :, 0:1] + degs_ref[1, 0, :, 0:1]
        din = degs_ref[0, 1, :, 0:1] + degs_ref[1, 1, :, 0:1]
        oisq = lax.rsqrt(jnp.maximum(dout, 1.0))
        iisq = lax.rsqrt(jnp.maximum(din, 1.0))
        oisq_ref[...] = oisq
        iisq_ref[...] = iisq
        y1_ref[...] = jnp.dot(x_ref[...] * oisq, w_ref[...],
                              preferred_element_type=jnp.float32)

    return pl.pallas_call(
        body,
        out_shape=(
            jax.ShapeDtypeStruct((NP, 1), jnp.float32),
            jax.ShapeDtypeStruct((NP, 1), jnp.float32),
            jax.ShapeDtypeStruct((NP, 64), jnp.float32),
        ),
    )(degs, feats, w_enc)


def _stage_b(parts, iisq, oisq, b_enc, w_emb):
    def body(p_ref, ii_ref, oi_ref, b_ref, w_ref, y2_ref):
        agg = p_ref[0] + p_ref[1]
        h = jnp.maximum(agg * ii_ref[...] + b_ref[...], 0.0)
        y2_ref[...] = jnp.dot(h * oi_ref[...], w_ref[...],
                              preferred_element_type=jnp.float32)

    return pl.pallas_call(
        body,
        out_shape=jax.ShapeDtypeStruct((NP, 32), jnp.float32),
    )(parts, iisq, oisq, b_enc, w_emb)


def _stage_c(parts, iisq, oisq, b_emb):
    def body(p_ref, ii_ref, oi_ref, b_ref, z_ref, y3_ref):
        z = (p_ref[0] + p_ref[1]) * ii_ref[...] + b_ref[...]
        z_ref[...] = z
        y3_ref[...] = z * oi_ref[...]

    return pl.pallas_call(
        body,
        out_shape=(
            jax.ShapeDtypeStruct((NP, 32), jnp.float32),
            jax.ShapeDtypeStruct((NP, 32), jnp.float32),
        ),
    )(parts, iisq, oisq, b_emb)


def _stage_d(parts, iisq, b_dec, w_dec):
    def body(p_ref, ii_ref, b_ref, w_ref, rec_ref):
        agg = (p_ref[0] + p_ref[1]) * ii_ref[...]
        rec_ref[...] = jnp.dot(agg, w_ref[...],
                               preferred_element_type=jnp.float32) + b_ref[...]

    return pl.pallas_call(
        body,
        out_shape=jax.ShapeDtypeStruct((NP, 128), jnp.float32),
    )(parts, iisq, b_dec, w_dec)


def kernel(features, edge_index, W_enc, b_enc, W_emb, b_emb, W_dec, b_dec):
    src = edge_index[0]
    dst = edge_index[1]
    pad = jnp.full((EP - EE,), NN, dtype=jnp.int32)
    src2 = jnp.concatenate([src, pad]).reshape(EP // CH, CH)
    dst2 = jnp.concatenate([dst, pad]).reshape(EP // CH, CH)

    feats = jnp.pad(features, ((0, NP - NN), (0, 0)))
    zeros16 = jnp.zeros((NP, 16), jnp.float32)
    ones16 = jnp.ones((CH, 16), jnp.float32)
    zeros64 = jnp.zeros((NP, 64), jnp.float32)
    zeros32 = jnp.zeros((NP, 32), jnp.float32)

    degs = _degree_kernel(src2, dst2, zeros16, ones16)
    oisq, iisq, y1 = _stage_a(degs, feats, W_enc)
    agg1 = _propagate(y1, src2, dst2, zeros64, 64)
    y2 = _stage_b(agg1, iisq, oisq, b_enc.reshape(1, -1), W_emb)
    agg2 = _propagate(y2, src2, dst2, zeros32, 32)
    z_full, y3 = _stage_c(agg2, iisq, oisq, b_emb.reshape(1, -1))
    agg3 = _propagate(y3, src2, dst2, zeros32, 32)
    recon_full = _stage_d(agg3, iisq, b_dec.reshape(1, -1), W_dec)

    return (z_full[:NN], recon_full[:NN])


# speedup vs baseline: 6.7308x; 1.2779x over previous
import functools

import jax
import jax.numpy as jnp
from jax import lax
from jax.experimental import pallas as pl
from jax.experimental.pallas import tpu as pltpu
from jax.experimental.pallas import tpu_sc as plsc

NN = 10000
NP = 10240
EE = 320000
NC = 2
NS = 16
CH = 128
KROW = 8
TILE_E = 10240
EP = TILE_E * NC * NS
OUTER = TILE_E // (KROW * CH)
ROWS_PER_TILE = NP // NS

_mesh = plsc.VectorSubcoreMesh(core_axis_name="c", subcore_axis_name="s")
_sc_params = pltpu.CompilerParams(use_tc_tiling_on_sc=False)


def _degree_kernel(src2, dst2, zeros16, ones16):
    @functools.partial(
        pl.kernel,
        mesh=_mesh,
        out_type=jax.ShapeDtypeStruct((NC, 2, NP, 16), jnp.float32),
        compiler_params=_sc_params,
        scratch_types=[
            pltpu.VMEM((KROW, CH), jnp.int32),
            pltpu.VMEM((KROW, CH), jnp.int32),
            pltpu.VMEM((CH, 16), jnp.float32),
            pltpu.VMEM_SHARED((NP, 16), jnp.float32),
            pltpu.VMEM_SHARED((NP, 16), jnp.float32),
        ],
    )
    def deg(src_hbm, dst_hbm, zeros_hbm, ones_hbm, out_hbm,
            sidx, didx, ones_v, acc_s, acc_d):
        cid = lax.axis_index("c")
        sid = lax.axis_index("s")
        wid = sid * NC + cid
        pltpu.sync_copy(ones_hbm, ones_v)
        sl = pl.ds(sid * ROWS_PER_TILE, ROWS_PER_TILE)
        pltpu.sync_copy(zeros_hbm.at[sl], acc_s.at[sl])
        pltpu.sync_copy(zeros_hbm.at[sl], acc_d.at[sl])
        plsc.subcore_barrier()
        row0 = wid * (TILE_E // CH)

        @pl.loop(0, OUTER)
        def _(i):
            base = row0 + i * KROW
            pltpu.sync_copy(src_hbm.at[pl.ds(base, KROW)], sidx)
            pltpu.sync_copy(dst_hbm.at[pl.ds(base, KROW)], didx)
            for j in range(KROW):
                pltpu.sync_copy(ones_v, acc_s.at[sidx.at[j]], add=True)
                pltpu.sync_copy(ones_v, acc_d.at[didx.at[j]], add=True)

        plsc.subcore_barrier()
        pltpu.sync_copy(acc_s.at[sl], out_hbm.at[cid, 0].at[sl])
        pltpu.sync_copy(acc_d.at[sl], out_hbm.at[cid, 1].at[sl])

    return deg(src2, dst2, zeros16, ones16)


NB = 8
LEAD = 4
CPT = TILE_E // CH


def _propagate(table, src2, dst2, zeros, F):
    @functools.partial(
        pl.kernel,
        mesh=_mesh,
        out_type=jax.ShapeDtypeStruct((NC, NP, F), jnp.float32),
        compiler_params=_sc_params,
        scratch_types=[
            pltpu.VMEM((CPT, CH), jnp.int32),
            pltpu.VMEM((CPT, CH), jnp.int32),
            pltpu.VMEM((NB, CH, F), jnp.float32),
            pltpu.VMEM_SHARED((NP, F), jnp.float32),
        ] + [pltpu.SemaphoreType.DMA] * (2 * NB),
    )
    def prop(table_hbm, src_hbm, dst_hbm, zeros_hbm, out_hbm,
             sidx, didx, rows, acc, *sems):
        gsem = sems[:NB]
        ssem = sems[NB:]
        cid = lax.axis_index("c")
        sid = lax.axis_index("s")
        wid = sid * NC + cid
        sl = pl.ds(sid * ROWS_PER_TILE, ROWS_PER_TILE)
        row0 = wid * CPT
        pltpu.sync_copy(src_hbm.at[pl.ds(row0, CPT)], sidx)
        pltpu.sync_copy(dst_hbm.at[pl.ds(row0, CPT)], didx)
        pltpu.sync_copy(zeros_hbm.at[sl], acc.at[sl])
        plsc.subcore_barrier()

        for b in range(LEAD):
            pltpu.async_copy(table_hbm.at[sidx.at[b]], rows.at[b], gsem[b])

        @pl.loop(0, CPT // NB)
        def _(i):
            c0 = i * NB
            for j in range(NB):
                c = c0 + j
                b4 = (j + LEAD) % NB

                @pl.when(c + LEAD < CPT)
                def _issue():
                    @pl.when(c >= LEAD)
                    def _drain():
                        pltpu.make_async_copy(
                            rows.at[b4], acc.at[didx.at[0]], ssem[b4]).wait()

                    pltpu.async_copy(
                        table_hbm.at[sidx.at[c + LEAD]], rows.at[b4], gsem[b4])

                pltpu.make_async_copy(
                    table_hbm.at[sidx.at[0]], rows.at[j], gsem[j]).wait()
                pltpu.async_copy(
                    rows.at[j], acc.at[didx.at[c]], ssem[j], add=True)

        for b in range(NB):
            pltpu.make_async_copy(
                rows.at[b], acc.at[didx.at[0]], ssem[b]).wait()

        plsc.subcore_barrier()
        pltpu.sync_copy(acc.at[sl], out_hbm.at[cid].at[sl])

    return prop(table, src2, dst2, zeros)


def _stage_a(degs, feats, w_enc):
    def body(degs_ref, x_ref, w_ref, oisq_ref, iisq_ref, y1_ref):
        dout = degs_ref[0, 0, :, 0:1] + degs_ref[1, 0, :, 0:1]
        din = degs_ref[0, 1, :, 0:1] + degs_ref[1, 1, :, 0:1]
        oisq = lax.rsqrt(jnp.maximum(dout, 1.0))
        iisq = lax.rsqrt(jnp.maximum(din, 1.0))
        oisq_ref[...] = oisq
        iisq_ref[...] = iisq
        y1_ref[...] = jnp.dot(x_ref[...] * oisq, w_ref[...],
                              preferred_element_type=jnp.float32)

    return pl.pallas_call(
        body,
        out_shape=(
            jax.ShapeDtypeStruct((NP, 1), jnp.float32),
            jax.ShapeDtypeStruct((NP, 1), jnp.float32),
            jax.ShapeDtypeStruct((NP, 64), jnp.float32),
        ),
    )(degs, feats, w_enc)


def _stage_b(parts, iisq, oisq, b_enc, w_emb):
    def body(p_ref, ii_ref, oi_ref, b_ref, w_ref, y2_ref):
        agg = p_ref[0] + p_ref[1]
        h = jnp.maximum(agg * ii_ref[...] + b_ref[...], 0.0)
        y2_ref[...] = jnp.dot(h * oi_ref[...], w_ref[...],
                              preferred_element_type=jnp.float32)

    return pl.pallas_call(
        body,
        out_shape=jax.ShapeDtypeStruct((NP, 32), jnp.float32),
    )(parts, iisq, oisq, b_enc, w_emb)


def _stage_c(parts, iisq, oisq, b_emb):
    def body(p_ref, ii_ref, oi_ref, b_ref, z_ref, y3_ref):
        z = (p_ref[0] + p_ref[1]) * ii_ref[...] + b_ref[...]
        z_ref[...] = z
        y3_ref[...] = z * oi_ref[...]

    return pl.pallas_call(
        body,
        out_shape=(
            jax.ShapeDtypeStruct((NP, 32), jnp.float32),
            jax.ShapeDtypeStruct((NP, 32), jnp.float32),
        ),
    )(parts, iisq, oisq, b_emb)


def _stage_d(parts, iisq, b_dec, w_dec):
    def body(p_ref, ii_ref, b_ref, w_ref, rec_ref):
        agg = (p_ref[0] + p_ref[1]) * ii_ref[...]
        rec_ref[...] = jnp.dot(agg, w_ref[...],
                               preferred_element_type=jnp.float32) + b_ref[...]

    return pl.pallas_call(
        body,
        out_shape=jax.ShapeDtypeStruct((NP, 128), jnp.float32),
    )(parts, iisq, b_dec, w_dec)


def kernel(features, edge_index, W_enc, b_enc, W_emb, b_emb, W_dec, b_dec):
    src = edge_index[0]
    dst = edge_index[1]
    pad = jnp.full((EP - EE,), NN, dtype=jnp.int32)
    src2 = jnp.concatenate([src, pad]).reshape(EP // CH, CH)
    dst2 = jnp.concatenate([dst, pad]).reshape(EP // CH, CH)

    feats = jnp.pad(features, ((0, NP - NN), (0, 0)))
    zeros16 = jnp.zeros((NP, 16), jnp.float32)
    ones16 = jnp.ones((CH, 16), jnp.float32)
    zeros64 = jnp.zeros((NP, 64), jnp.float32)
    zeros32 = jnp.zeros((NP, 32), jnp.float32)

    degs = _degree_kernel(src2, dst2, zeros16, ones16)
    oisq, iisq, y1 = _stage_a(degs, feats, W_enc)
    agg1 = _propagate(y1, src2, dst2, zeros64, 64)
    y2 = _stage_b(agg1, iisq, oisq, b_enc.reshape(1, -1), W_emb)
    agg2 = _propagate(y2, src2, dst2, zeros32, 32)
    z_full, y3 = _stage_c(agg2, iisq, oisq, b_emb.reshape(1, -1))
    agg3 = _propagate(y3, src2, dst2, zeros32, 32)
    recon_full = _stage_d(agg3, iisq, b_dec.reshape(1, -1), W_dec)

    return (z_full[:NN], recon_full[:NN])


# speedup vs baseline: 12.6630x; 1.8814x over previous
import functools

import jax
import jax.numpy as jnp
from jax import lax
from jax.experimental import pallas as pl
from jax.experimental.pallas import tpu as pltpu
from jax.experimental.pallas import tpu_sc as plsc

NN = 10000
NP = 10240
EE = 320000
NC = 2
NS = 16
CH = 128
KROW = 8
TILE_E = 20480
EP = TILE_E * NS
ROWS_PER_TILE = NP // NS
NB = 8
LEAD = 4
CPT = TILE_E // CH

_mesh = plsc.VectorSubcoreMesh(core_axis_name="c", subcore_axis_name="s")
_sc_params = pltpu.CompilerParams(use_tc_tiling_on_sc=False)


def _degree_kernel(idx3, zeros16, ones16):
    @functools.partial(
        pl.kernel,
        mesh=_mesh,
        out_type=jax.ShapeDtypeStruct((NC, NP, 16), jnp.float32),
        compiler_params=_sc_params,
        scratch_types=[
            pltpu.VMEM((KROW, CH), jnp.int32),
            pltpu.VMEM((CH, 16), jnp.float32),
            pltpu.VMEM_SHARED((NP, 16), jnp.float32),
        ],
    )
    def deg(idx_hbm, zeros_hbm, ones_hbm, out_hbm, bidx, ones_v, acc):
        cid = lax.axis_index("c")
        sid = lax.axis_index("s")
        pltpu.sync_copy(ones_hbm, ones_v)
        sl = pl.ds(sid * ROWS_PER_TILE, ROWS_PER_TILE)
        pltpu.sync_copy(zeros_hbm.at[sl], acc.at[sl])
        plsc.subcore_barrier()
        row0 = sid * CPT

        @pl.loop(0, CPT // KROW)
        def _(i):
            pltpu.sync_copy(idx_hbm.at[cid].at[pl.ds(row0 + i * KROW, KROW)],
                            bidx)
            for j in range(KROW):
                pltpu.sync_copy(ones_v, acc.at[bidx.at[j]], add=True)

        plsc.subcore_barrier()
        pltpu.sync_copy(acc.at[sl], out_hbm.at[cid].at[sl])

    return deg(idx3, zeros16, ones16)


def _propagate(table2, src2, dst2, zeros, F2):
    @functools.partial(
        pl.kernel,
        mesh=_mesh,
        out_type=jax.ShapeDtypeStruct((NC, NP, F2), jnp.float32),
        compiler_params=_sc_params,
        scratch_types=[
            pltpu.VMEM((CPT, CH), jnp.int32),
            pltpu.VMEM((CPT, CH), jnp.int32),
            pltpu.VMEM((NB, CH, F2), jnp.float32),
            pltpu.VMEM_SHARED((NP, F2), jnp.float32),
            pltpu.VMEM_SHARED((NP, F2), jnp.float32),
        ] + [pltpu.SemaphoreType.DMA] * (2 * NB),
    )
    def prop(table_hbm, src_hbm, dst_hbm, zeros_hbm, out_hbm,
             sidx, didx, rows, acc, table_sh, *sems):
        gsem = sems[:NB]
        ssem = sems[NB:]
        cid = lax.axis_index("c")
        sid = lax.axis_index("s")
        sl = pl.ds(sid * ROWS_PER_TILE, ROWS_PER_TILE)
        row0 = sid * CPT
        pltpu.sync_copy(src_hbm.at[pl.ds(row0, CPT)], sidx)
        pltpu.sync_copy(dst_hbm.at[pl.ds(row0, CPT)], didx)
        pltpu.sync_copy(zeros_hbm.at[sl], acc.at[sl])
        pltpu.sync_copy(table_hbm.at[cid].at[sl], table_sh.at[sl])
        plsc.subcore_barrier()

        for b in range(LEAD):
            pltpu.async_copy(table_sh.at[sidx.at[b]], rows.at[b], gsem[b])

        @pl.loop(0, CPT // NB)
        def _(i):
            c0 = i * NB
            for j in range(NB):
                c = c0 + j
                b4 = (j + LEAD) % NB

                @pl.when(c + LEAD < CPT)
                def _issue():
                    @pl.when(c >= LEAD)
                    def _drain():
                        pltpu.make_async_copy(
                            rows.at[b4], acc.at[didx.at[0]], ssem[b4]).wait()

                    pltpu.async_copy(
                        table_sh.at[sidx.at[c + LEAD]], rows.at[b4], gsem[b4])

                pltpu.make_async_copy(
                    table_sh.at[sidx.at[0]], rows.at[j], gsem[j]).wait()
                pltpu.async_copy(
                    rows.at[j], acc.at[didx.at[c]], ssem[j], add=True)

        for b in range(NB):
            pltpu.make_async_copy(
                rows.at[b], acc.at[didx.at[0]], ssem[b]).wait()

        plsc.subcore_barrier()
        pltpu.sync_copy(acc.at[sl], out_hbm.at[cid].at[sl])

    return prop(table2, src2, dst2, zeros)


def _matmul_xw(feats, w_enc):
    def body(x_ref, w_ref, xw_ref):
        xw_ref[...] = jnp.dot(x_ref[...], w_ref[...],
                              preferred_element_type=jnp.float32)

    return pl.pallas_call(
        body,
        out_shape=jax.ShapeDtypeStruct((NP, 64), jnp.float32),
    )(feats, w_enc)


def _stage_a(degs, xw):
    def body(degs_ref, xw_ref, oisq_ref, iisq_ref, y1_ref):
        dout = degs_ref[0, :, 0:1]
        din = degs_ref[1, :, 0:1]
        oisq = lax.rsqrt(jnp.maximum(dout, 1.0))
        iisq = lax.rsqrt(jnp.maximum(din, 1.0))
        oisq_ref[...] = oisq
        iisq_ref[...] = iisq
        y1 = xw_ref[...] * oisq
        y1_ref[0] = y1[:, :32]
        y1_ref[1] = y1[:, 32:]

    return pl.pallas_call(
        body,
        out_shape=(
            jax.ShapeDtypeStruct((NP, 1), jnp.float32),
            jax.ShapeDtypeStruct((NP, 1), jnp.float32),
            jax.ShapeDtypeStruct((2, NP, 32), jnp.float32),
        ),
    )(degs, xw)


def _stage_b(parts, iisq, oisq, b_enc, w_emb):
    def body(p_ref, ii_ref, oi_ref, b_ref, w_ref, y2_ref):
        agg = jnp.concatenate([p_ref[0], p_ref[1]], axis=1)
        h = jnp.maximum(agg * ii_ref[...] + b_ref[...], 0.0)
        y2 = jnp.dot(h * oi_ref[...], w_ref[...],
                     preferred_element_type=jnp.float32)
        y2_ref[0] = y2[:, :16]
        y2_ref[1] = y2[:, 16:]

    return pl.pallas_call(
        body,
        out_shape=jax.ShapeDtypeStruct((2, NP, 16), jnp.float32),
    )(parts, iisq, oisq, b_enc, w_emb)


def _stage_c(parts, iisq, oisq, b_emb):
    def body(p_ref, ii_ref, oi_ref, b_ref, z_ref, y3_ref):
        z = jnp.concatenate([p_ref[0], p_ref[1]], axis=1) * ii_ref[...] \
            + b_ref[...]
        z_ref[...] = z
        y3 = z * oi_ref[...]
        y3_ref[0] = y3[:, :16]
        y3_ref[1] = y3[:, 16:]

    return pl.pallas_call(
        body,
        out_shape=(
            jax.ShapeDtypeStruct((NP, 32), jnp.float32),
            jax.ShapeDtypeStruct((2, NP, 16), jnp.float32),
        ),
    )(parts, iisq, oisq, b_emb)


def _stage_d(parts, iisq, b_dec, w_dec):
    def body(p_ref, ii_ref, b_ref, w_ref, rec_ref):
        agg = jnp.concatenate([p_ref[0], p_ref[1]], axis=1) * ii_ref[...]
        rec_ref[...] = jnp.dot(agg, w_ref[...],
                               preferred_element_type=jnp.float32) + b_ref[...]

    return pl.pallas_call(
        body,
        out_shape=jax.ShapeDtypeStruct((NP, 128), jnp.float32),
    )(parts, iisq, b_dec, w_dec)


def kernel(features, edge_index, W_enc, b_enc, W_emb, b_emb, W_dec, b_dec):
    src = edge_index[0]
    dst = edge_index[1]
    pad = jnp.full((EP - EE,), NN, dtype=jnp.int32)
    src2 = jnp.concatenate([src, pad]).reshape(EP // CH, CH)
    dst2 = jnp.concatenate([dst, pad]).reshape(EP // CH, CH)
    idx3 = jnp.stack([src2, dst2])

    feats = jnp.pad(features, ((0, NP - NN), (0, 0)))
    zeros16 = jnp.zeros((NP, 16), jnp.float32)
    ones16 = jnp.ones((CH, 16), jnp.float32)
    zeros32 = jnp.zeros((NP, 32), jnp.float32)

    xw = _matmul_xw(feats, W_enc)
    degs = _degree_kernel(idx3, zeros16, ones16)
    oisq, iisq, y1 = _stage_a(degs, xw)
    agg1 = _propagate(y1, src2, dst2, zeros32, 32)
    y2 = _stage_b(agg1, iisq, oisq, b_enc.reshape(1, -1), W_emb)
    agg2 = _propagate(y2, src2, dst2, zeros16, 16)
    z_full, y3 = _stage_c(agg2, iisq, oisq, b_emb.reshape(1, -1))
    agg3 = _propagate(y3, src2, dst2, zeros16, 16)
    recon_full = _stage_d(agg3, iisq, b_dec.reshape(1, -1), W_dec)

    return (z_full[:NN], recon_full[:NN])


# speedup vs baseline: 14.1845x; 1.1202x over previous
import functools

import jax
import jax.numpy as jnp
from jax import lax
from jax.experimental import pallas as pl
from jax.experimental.pallas import tpu as pltpu
from jax.experimental.pallas import tpu_sc as plsc

NN = 10000
NP = 10240
EE = 320000
NC = 2
NS = 16
CH = 128
KROW = 8
TILE_E = 20480
EP = TILE_E * NS
ROWS_PER_TILE = NP // NS
NB = 8
LEAD = 4
CPT = TILE_E // CH

_mesh = plsc.VectorSubcoreMesh(core_axis_name="c", subcore_axis_name="s")
_sc_params = pltpu.CompilerParams(use_tc_tiling_on_sc=False)


def _degree_kernel(idx3, zeros16, ones16):
    @functools.partial(
        pl.kernel,
        mesh=_mesh,
        out_type=jax.ShapeDtypeStruct((NC, NP, 128), jnp.float32),
        compiler_params=_sc_params,
        scratch_types=[
            pltpu.VMEM((KROW, CH), jnp.int32),
            pltpu.VMEM((CH, 16), jnp.float32),
            pltpu.VMEM_SHARED((NP, 16), jnp.float32),
        ],
    )
    def deg(idx_hbm, zeros_hbm, ones_hbm, out_hbm, bidx, ones_v, acc):
        cid = lax.axis_index("c")
        sid = lax.axis_index("s")
        pltpu.sync_copy(ones_hbm, ones_v)
        sl = pl.ds(sid * ROWS_PER_TILE, ROWS_PER_TILE)
        pltpu.sync_copy(zeros_hbm.at[sl], acc.at[sl])
        plsc.subcore_barrier()
        row0 = sid * CPT

        @pl.loop(0, CPT // KROW)
        def _(i):
            pltpu.sync_copy(idx_hbm.at[cid].at[pl.ds(row0 + i * KROW, KROW)],
                            bidx)
            for j in range(KROW):
                pltpu.sync_copy(ones_v, acc.at[bidx.at[j]], add=True)

        plsc.subcore_barrier()
        pltpu.sync_copy(acc.at[sl], out_hbm.at[cid].at[sl, pl.ds(0, 16)])

    return deg(idx3, zeros16, ones16)


def _propagate(table2, src2, dst2, zeros, F2):
    @functools.partial(
        pl.kernel,
        mesh=_mesh,
        out_type=jax.ShapeDtypeStruct((NC, NP, 128), jnp.float32),
        compiler_params=_sc_params,
        scratch_types=[
            pltpu.VMEM((CPT, CH), jnp.int32),
            pltpu.VMEM((CPT, CH), jnp.int32),
            pltpu.VMEM((NB, CH, F2), jnp.float32),
            pltpu.VMEM_SHARED((NP, F2), jnp.float32),
            pltpu.VMEM_SHARED((NP, F2), jnp.float32),
        ] + [pltpu.SemaphoreType.DMA] * (2 * NB),
    )
    def prop(table_hbm, src_hbm, dst_hbm, zeros_hbm, out_hbm,
             sidx, didx, rows, acc, table_sh, *sems):
        gsem = sems[:NB]
        ssem = sems[NB:]
        cid = lax.axis_index("c")
        sid = lax.axis_index("s")
        sl = pl.ds(sid * ROWS_PER_TILE, ROWS_PER_TILE)
        row0 = sid * CPT
        pltpu.sync_copy(src_hbm.at[pl.ds(row0, CPT)], sidx)
        pltpu.sync_copy(dst_hbm.at[pl.ds(row0, CPT)], didx)
        pltpu.sync_copy(zeros_hbm.at[sl], acc.at[sl])
        pltpu.sync_copy(table_hbm.at[cid].at[sl, pl.ds(0, F2)], table_sh.at[sl])
        plsc.subcore_barrier()

        for b in range(LEAD):
            pltpu.async_copy(table_sh.at[sidx.at[b]], rows.at[b], gsem[b])

        @pl.loop(0, CPT // NB)
        def _(i):
            c0 = i * NB
            for j in range(NB):
                c = c0 + j
                b4 = (j + LEAD) % NB

                @pl.when(c + LEAD < CPT)
                def _issue():
                    @pl.when(c >= LEAD)
                    def _drain():
                        pltpu.make_async_copy(
                            rows.at[b4], acc.at[didx.at[0]], ssem[b4]).wait()

                    pltpu.async_copy(
                        table_sh.at[sidx.at[c + LEAD]], rows.at[b4], gsem[b4])

                pltpu.make_async_copy(
                    table_sh.at[sidx.at[0]], rows.at[j], gsem[j]).wait()
                pltpu.async_copy(
                    rows.at[j], acc.at[didx.at[c]], ssem[j], add=True)

        for b in range(NB):
            pltpu.make_async_copy(
                rows.at[b], acc.at[didx.at[0]], ssem[b]).wait()

        plsc.subcore_barrier()
        pltpu.sync_copy(acc.at[sl], out_hbm.at[cid].at[sl, pl.ds(0, F2)])

    return prop(table2, src2, dst2, zeros)


def _matmul_xw(feats, w_enc):
    def body(x_ref, w_ref, xw_ref):
        xw_ref[...] = jnp.dot(x_ref[...], w_ref[...],
                              preferred_element_type=jnp.float32)

    return pl.pallas_call(
        body,
        out_shape=jax.ShapeDtypeStruct((NP, 64), jnp.float32),
    )(feats, w_enc)


def _stage_a(degs, xw):
    def body(degs_ref, xw_ref, oisq_ref, iisq_ref, y1_ref):
        dout = degs_ref[0, :, 0:1]
        din = degs_ref[1, :, 0:1]
        oisq = lax.rsqrt(jnp.maximum(dout, 1.0))
        iisq = lax.rsqrt(jnp.maximum(din, 1.0))
        oisq_ref[...] = oisq
        iisq_ref[...] = iisq
        y1 = xw_ref[...] * oisq
        y1_ref[0, :, 0:32] = y1[:, :32]
        y1_ref[1, :, 0:32] = y1[:, 32:]

    return pl.pallas_call(
        body,
        out_shape=(
            jax.ShapeDtypeStruct((NP, 1), jnp.float32),
            jax.ShapeDtypeStruct((NP, 1), jnp.float32),
            jax.ShapeDtypeStruct((2, NP, 128), jnp.float32),
        ),
    )(degs, xw)


def _stage_b(parts, iisq, oisq, b_enc, w_emb):
    def body(p_ref, ii_ref, oi_ref, b_ref, w_ref, y2_ref):
        agg = jnp.concatenate([p_ref[0, :, 0:32], p_ref[1, :, 0:32]], axis=1)
        h = jnp.maximum(agg * ii_ref[...] + b_ref[...], 0.0)
        y2 = jnp.dot(h * oi_ref[...], w_ref[...],
                     preferred_element_type=jnp.float32)
        y2_ref[0, :, 0:16] = y2[:, :16]
        y2_ref[1, :, 0:16] = y2[:, 16:]

    return pl.pallas_call(
        body,
        out_shape=jax.ShapeDtypeStruct((2, NP, 128), jnp.float32),
    )(parts, iisq, oisq, b_enc, w_emb)


def _stage_c(parts, iisq, oisq, b_emb):
    def body(p_ref, ii_ref, oi_ref, b_ref, z_ref, y3_ref):
        z = jnp.concatenate([p_ref[0, :, 0:16], p_ref[1, :, 0:16]], axis=1) \
            * ii_ref[...] + b_ref[...]
        z_ref[...] = z
        y3 = z * oi_ref[...]
        y3_ref[0, :, 0:16] = y3[:, :16]
        y3_ref[1, :, 0:16] = y3[:, 16:]

    return pl.pallas_call(
        body,
        out_shape=(
            jax.ShapeDtypeStruct((NP, 32), jnp.float32),
            jax.ShapeDtypeStruct((2, NP, 128), jnp.float32),
        ),
    )(parts, iisq, oisq, b_emb)


def _stage_d(parts, iisq, b_dec, w_dec):
    def body(p_ref, ii_ref, b_ref, w_ref, rec_ref):
        agg = jnp.concatenate([p_ref[0, :, 0:16], p_ref[1, :, 0:16]],
                              axis=1) * ii_ref[...]
        rec_ref[...] = jnp.dot(agg, w_ref[...],
                               preferred_element_type=jnp.float32) + b_ref[...]

    return pl.pallas_call(
        body,
        out_shape=jax.ShapeDtypeStruct((NP, 128), jnp.float32),
    )(parts, iisq, b_dec, w_dec)


def kernel(features, edge_index, W_enc, b_enc, W_emb, b_emb, W_dec, b_dec):
    src = edge_index[0]
    dst = edge_index[1]
    pad = jnp.full((EP - EE,), NN, dtype=jnp.int32)
    src2 = jnp.concatenate([src, pad]).reshape(EP // CH, CH)
    dst2 = jnp.concatenate([dst, pad]).reshape(EP // CH, CH)
    idx3 = jnp.stack([src2, dst2])

    feats = jnp.pad(features, ((0, NP - NN), (0, 0)))
    zeros16 = jnp.zeros((NP, 16), jnp.float32)
    ones16 = jnp.ones((CH, 16), jnp.float32)
    zeros32 = jnp.zeros((NP, 32), jnp.float32)

    xw = _matmul_xw(feats, W_enc)
    degs = _degree_kernel(idx3, zeros16, ones16)
    oisq, iisq, y1 = _stage_a(degs, xw)
    agg1 = _propagate(y1, src2, dst2, zeros32, 32)
    y2 = _stage_b(agg1, iisq, oisq, b_enc.reshape(1, -1), W_emb)
    agg2 = _propagate(y2, src2, dst2, zeros16, 16)
    z_full, y3 = _stage_c(agg2, iisq, oisq, b_emb.reshape(1, -1))
    agg3 = _propagate(y3, src2, dst2, zeros16, 16)
    recon_full = _stage_d(agg3, iisq, b_dec.reshape(1, -1), W_dec)

    return (z_full[:NN], recon_full[:NN])


# speedup vs baseline: 15.3625x; 1.0830x over previous
import functools

import jax
import jax.numpy as jnp
from jax import lax
from jax.experimental import pallas as pl
from jax.experimental.pallas import tpu as pltpu
from jax.experimental.pallas import tpu_sc as plsc

NN = 10000
NP = 10240
EE = 320000
NC = 2
NS = 16
CH = 128
KROW = 8
TILE_E = 20480
EP = TILE_E * NS
ROWS_PER_TILE = NP // NS
NB = 8
LEAD = 4
CPT = TILE_E // CH

_mesh = plsc.VectorSubcoreMesh(core_axis_name="c", subcore_axis_name="s")
_sc_params = pltpu.CompilerParams(use_tc_tiling_on_sc=False)


def _degree_kernel(idx3, zeros16, ones16):
    @functools.partial(
        pl.kernel,
        mesh=_mesh,
        out_type=jax.ShapeDtypeStruct((NC, NP, 128), jnp.float32),
        compiler_params=_sc_params,
        scratch_types=[
            pltpu.VMEM((CPT, CH), jnp.int32),
            pltpu.VMEM((CH, 16), jnp.float32),
            pltpu.VMEM_SHARED((NP, 16), jnp.float32),
        ] + [pltpu.SemaphoreType.DMA] * NB,
    )
    def deg(idx_hbm, zeros_hbm, ones_hbm, out_hbm, bidx, ones_v, acc, *sems):
        cid = lax.axis_index("c")
        sid = lax.axis_index("s")
        pltpu.sync_copy(ones_hbm, ones_v)
        sl = pl.ds(sid * ROWS_PER_TILE, ROWS_PER_TILE)
        pltpu.sync_copy(zeros_hbm.at[sl], acc.at[sl])
        pltpu.sync_copy(idx_hbm.at[cid].at[pl.ds(sid * CPT, CPT)], bidx)
        plsc.subcore_barrier()

        @pl.loop(0, CPT // NB)
        def _(i):
            c0 = i * NB
            for j in range(NB):
                @pl.when(c0 + j >= NB)
                def _drain():
                    pltpu.make_async_copy(
                        ones_v, acc.at[bidx.at[0]], sems[j]).wait()

                pltpu.async_copy(
                    ones_v, acc.at[bidx.at[c0 + j]], sems[j], add=True)

        for j in range(NB):
            pltpu.make_async_copy(ones_v, acc.at[bidx.at[0]], sems[j]).wait()

        plsc.subcore_barrier()
        pltpu.sync_copy(acc.at[sl], out_hbm.at[cid].at[sl, pl.ds(0, 16)])

    return deg(idx3, zeros16, ones16)


def _propagate(table2, idx3, zeros, F2):
    @functools.partial(
        pl.kernel,
        mesh=_mesh,
        out_type=jax.ShapeDtypeStruct((NC, NP, 128), jnp.float32),
        compiler_params=_sc_params,
        scratch_types=[
            pltpu.VMEM((CPT, CH), jnp.int32),
            pltpu.VMEM((CPT, CH), jnp.int32),
            pltpu.VMEM((NB, CH, F2), jnp.float32),
            pltpu.VMEM_SHARED((NP, F2), jnp.float32),
            pltpu.VMEM_SHARED((NP, F2), jnp.float32),
        ] + [pltpu.SemaphoreType.DMA] * (2 * NB),
    )
    def prop(table_hbm, idx_hbm, zeros_hbm, out_hbm,
             sidx, didx, rows, acc, table_sh, *sems):
        gsem = sems[:NB]
        ssem = sems[NB:]
        cid = lax.axis_index("c")
        sid = lax.axis_index("s")
        sl = pl.ds(sid * ROWS_PER_TILE, ROWS_PER_TILE)
        row0 = sid * CPT
        pltpu.sync_copy(idx_hbm.at[0].at[pl.ds(row0, CPT)], sidx)
        pltpu.sync_copy(idx_hbm.at[1].at[pl.ds(row0, CPT)], didx)
        pltpu.sync_copy(zeros_hbm.at[sl], acc.at[sl])
        pltpu.sync_copy(table_hbm.at[cid].at[sl, pl.ds(0, F2)], table_sh.at[sl])
        plsc.subcore_barrier()

        for b in range(LEAD):
            pltpu.async_copy(table_sh.at[sidx.at[b]], rows.at[b], gsem[b])

        @pl.loop(0, CPT // NB)
        def _(i):
            c0 = i * NB
            for j in range(NB):
                c = c0 + j
                b4 = (j + LEAD) % NB

                @pl.when(c + LEAD < CPT)
                def _issue():
                    @pl.when(c >= LEAD)
                    def _drain():
                        pltpu.make_async_copy(
                            rows.at[b4], acc.at[didx.at[0]], ssem[b4]).wait()

                    pltpu.async_copy(
                        table_sh.at[sidx.at[c + LEAD]], rows.at[b4], gsem[b4])

                pltpu.make_async_copy(
                    table_sh.at[sidx.at[0]], rows.at[j], gsem[j]).wait()
                pltpu.async_copy(
                    rows.at[j], acc.at[didx.at[c]], ssem[j], add=True)

        for b in range(NB):
            pltpu.make_async_copy(
                rows.at[b], acc.at[didx.at[0]], ssem[b]).wait()

        plsc.subcore_barrier()
        pltpu.sync_copy(acc.at[sl], out_hbm.at[cid].at[sl, pl.ds(0, F2)])

    return prop(table2, idx3, zeros)


def _matmul_xw(feats, w_enc):
    def body(x_ref, w_ref, xw_ref):
        xw_ref[...] = jnp.dot(x_ref[...], w_ref[...],
                              preferred_element_type=jnp.float32)

    return pl.pallas_call(
        body,
        out_shape=jax.ShapeDtypeStruct((NP, 64), jnp.float32),
    )(feats, w_enc)


def _stage_a(degs, xw):
    def body(degs_ref, xw_ref, oisq_ref, iisq_ref, y1_ref):
        dout = degs_ref[0, :, 0:1]
        din = degs_ref[1, :, 0:1]
        oisq = lax.rsqrt(jnp.maximum(dout, 1.0))
        iisq = lax.rsqrt(jnp.maximum(din, 1.0))
        oisq_ref[...] = oisq
        iisq_ref[...] = iisq
        y1 = xw_ref[...] * oisq
        y1_ref[0, :, 0:32] = y1[:, :32]
        y1_ref[1, :, 0:32] = y1[:, 32:]

    return pl.pallas_call(
        body,
        out_shape=(
            jax.ShapeDtypeStruct((NP, 1), jnp.float32),
            jax.ShapeDtypeStruct((NP, 1), jnp.float32),
            jax.ShapeDtypeStruct((2, NP, 128), jnp.float32),
        ),
    )(degs, xw)


def _stage_b(parts, iisq, oisq, b_enc, w_emb):
    def body(p_ref, ii_ref, oi_ref, b_ref, w_ref, y2_ref):
        agg = jnp.concatenate([p_ref[0, :, 0:32], p_ref[1, :, 0:32]], axis=1)
        h = jnp.maximum(agg * ii_ref[...] + b_ref[...], 0.0)
        y2 = jnp.dot(h * oi_ref[...], w_ref[...],
                     preferred_element_type=jnp.float32)
        y2_ref[0, :, 0:16] = y2[:, :16]
        y2_ref[1, :, 0:16] = y2[:, 16:]

    return pl.pallas_call(
        body,
        out_shape=jax.ShapeDtypeStruct((2, NP, 128), jnp.float32),
    )(parts, iisq, oisq, b_enc, w_emb)


def _stage_c(parts, iisq, oisq, b_emb):
    def body(p_ref, ii_ref, oi_ref, b_ref, z_ref, y3_ref):
        z = jnp.concatenate([p_ref[0, :, 0:16], p_ref[1, :, 0:16]], axis=1) \
            * ii_ref[...] + b_ref[...]
        z_ref[...] = z
        y3 = z * oi_ref[...]
        y3_ref[0, :, 0:16] = y3[:, :16]
        y3_ref[1, :, 0:16] = y3[:, 16:]

    return pl.pallas_call(
        body,
        out_shape=(
            jax.ShapeDtypeStruct((NP, 32), jnp.float32),
            jax.ShapeDtypeStruct((2, NP, 128), jnp.float32),
        ),
    )(parts, iisq, oisq, b_emb)


def _stage_d(parts, iisq, b_dec, w_dec):
    def body(p_ref, ii_ref, b_ref, w_ref, rec_ref):
        agg = jnp.concatenate([p_ref[0, :, 0:16], p_ref[1, :, 0:16]],
                              axis=1) * ii_ref[...]
        rec_ref[...] = jnp.dot(agg, w_ref[...],
                               preferred_element_type=jnp.float32) + b_ref[...]

    return pl.pallas_call(
        body,
        out_shape=jax.ShapeDtypeStruct((NP, 128), jnp.float32),
    )(parts, iisq, b_dec, w_dec)


def kernel(features, edge_index, W_enc, b_enc, W_emb, b_emb, W_dec, b_dec):
    idx3 = jnp.pad(edge_index.reshape(2, EE // CH, CH),
                   ((0, 0), (0, (EP - EE) // CH), (0, 0)),
                   constant_values=NN)

    feats = jnp.pad(features, ((0, NP - NN), (0, 0)))
    zeros16 = jnp.zeros((NP, 16), jnp.float32)
    ones16 = jnp.ones((CH, 16), jnp.float32)
    zeros32 = jnp.zeros((NP, 32), jnp.float32)

    xw = _matmul_xw(feats, W_enc)
    degs = _degree_kernel(idx3, zeros16, ones16)
    oisq, iisq, y1 = _stage_a(degs, xw)
    agg1 = _propagate(y1, idx3, zeros32, 32)
    y2 = _stage_b(agg1, iisq, oisq, b_enc.reshape(1, -1), W_emb)
    agg2 = _propagate(y2, idx3, zeros16, 16)
    z_full, y3 = _stage_c(agg2, iisq, oisq, b_emb.reshape(1, -1))
    agg3 = _propagate(y3, idx3, zeros16, 16)
    recon_full = _stage_d(agg3, iisq, b_dec.reshape(1, -1), W_dec)

    return (z_full[:NN], recon_full[:NN])


# speedup vs baseline: 15.5071x; 1.0094x over previous
import functools

import jax
import jax.numpy as jnp
from jax import lax
from jax.experimental import pallas as pl
from jax.experimental.pallas import tpu as pltpu
from jax.experimental.pallas import tpu_sc as plsc

NN = 10000
NP = 10240
EE = 320000
NC = 2
NS = 16
CH = 128
KROW = 8
TILE_E = 20480
EP = TILE_E * NS
ROWS_PER_TILE = NP // NS
NB = 8
LEAD = 4
CPT = TILE_E // CH

_mesh = plsc.VectorSubcoreMesh(core_axis_name="c", subcore_axis_name="s")
_sc_params = pltpu.CompilerParams(use_tc_tiling_on_sc=False)


def _degree_kernel(idx3, zeros16, ones16):
    @functools.partial(
        pl.kernel,
        mesh=_mesh,
        out_type=jax.ShapeDtypeStruct((NC, NP, 128), jnp.float32),
        compiler_params=_sc_params,
        scratch_types=[
            pltpu.VMEM((CPT, CH), jnp.int32),
            pltpu.VMEM((CH, 16), jnp.float32),
            pltpu.VMEM_SHARED((NP, 16), jnp.float32),
        ] + [pltpu.SemaphoreType.DMA] * NB,
    )
    def deg(idx_hbm, zeros_hbm, ones_hbm, out_hbm, bidx, ones_v, acc, *sems):
        cid = lax.axis_index("c")
        sid = lax.axis_index("s")
        pltpu.sync_copy(ones_hbm, ones_v)
        sl = pl.ds(sid * ROWS_PER_TILE, ROWS_PER_TILE)
        pltpu.sync_copy(zeros_hbm.at[sl], acc.at[sl])
        pltpu.sync_copy(idx_hbm.at[cid].at[pl.ds(sid * CPT, CPT)], bidx)
        plsc.subcore_barrier()

        @pl.loop(0, CPT // NB)
        def _(i):
            c0 = i * NB
            for j in range(NB):
                @pl.when(c0 + j >= NB)
                def _drain():
                    pltpu.make_async_copy(
                        ones_v, acc.at[bidx.at[0]], sems[j]).wait()

                pltpu.async_copy(
                    ones_v, acc.at[bidx.at[c0 + j]], sems[j], add=True)

        for j in range(NB):
            pltpu.make_async_copy(ones_v, acc.at[bidx.at[0]], sems[j]).wait()

        plsc.subcore_barrier()
        pltpu.sync_copy(acc.at[sl], out_hbm.at[cid].at[sl, pl.ds(0, 16)])

    return deg(idx3, zeros16, ones16)


def _propagate(table2, idx3, zeros, F2):
    @functools.partial(
        pl.kernel,
        mesh=_mesh,
        out_type=jax.ShapeDtypeStruct((NC, NP, 128), jnp.float32),
        compiler_params=_sc_params,
        scratch_types=[
            pltpu.VMEM((CPT, CH), jnp.int32),
            pltpu.VMEM((CPT, CH), jnp.int32),
            pltpu.VMEM((NB, CH, F2), jnp.float32),
            pltpu.VMEM_SHARED((NP, F2), jnp.float32),
            pltpu.VMEM_SHARED((NP, F2), jnp.float32),
        ] + [pltpu.SemaphoreType.DMA] * (2 * NB),
    )
    def prop(table_hbm, idx_hbm, zeros_hbm, out_hbm,
             sidx, didx, rows, acc, table_sh, *sems):
        gsem = sems[:NB]
        ssem = sems[NB:]
        cid = lax.axis_index("c")
        sid = lax.axis_index("s")
        sl = pl.ds(sid * ROWS_PER_TILE, ROWS_PER_TILE)
        row0 = sid * CPT
        pltpu.sync_copy(idx_hbm.at[0].at[pl.ds(row0, CPT)], sidx)
        pltpu.sync_copy(idx_hbm.at[1].at[pl.ds(row0, CPT)], didx)
        pltpu.sync_copy(zeros_hbm.at[sl], acc.at[sl])
        pltpu.sync_copy(table_hbm.at[cid].at[sl, pl.ds(0, F2)], table_sh.at[sl])
        plsc.subcore_barrier()

        for b in range(LEAD):
            pltpu.async_copy(table_sh.at[sidx.at[b]], rows.at[b], gsem[b])

        @pl.loop(0, CPT // NB)
        def _(i):
            c0 = i * NB
            for j in range(NB):
                c = c0 + j
                b4 = (j + LEAD) % NB

                @pl.when(c + LEAD < CPT)
                def _issue():
                    @pl.when(c >= LEAD)
                    def _drain():
                        pltpu.make_async_copy(
                            rows.at[b4], acc.at[didx.at[0]], ssem[b4]).wait()

                    pltpu.async_copy(
                        table_sh.at[sidx.at[c + LEAD]], rows.at[b4], gsem[b4])

                pltpu.make_async_copy(
                    table_sh.at[sidx.at[0]], rows.at[j], gsem[j]).wait()
                pltpu.async_copy(
                    rows.at[j], acc.at[didx.at[c]], ssem[j], add=True)

        for b in range(NB):
            pltpu.make_async_copy(
                rows.at[b], acc.at[didx.at[0]], ssem[b]).wait()

        plsc.subcore_barrier()
        pltpu.sync_copy(acc.at[sl], out_hbm.at[cid].at[sl, pl.ds(0, F2)])

    return prop(table2, idx3, zeros)


def _matmul_xw(feats, w_enc):
    def body(x_ref, w_ref, xw_ref):
        xw_ref[...] = jnp.dot(x_ref[...], w_ref[...],
                              preferred_element_type=jnp.float32)

    return pl.pallas_call(
        body,
        grid=(NR,),
        in_specs=[
            pl.BlockSpec((BR, 128), lambda i: (i, 0)),
            pl.BlockSpec((128, 64), lambda i: (0, 0)),
        ],
        out_specs=pl.BlockSpec((BR, 64), lambda i: (i, 0)),
        out_shape=jax.ShapeDtypeStruct((NP, 64), jnp.float32),
    )(feats, w_enc)


NR = 8
BR = NP // NR


def _stage_a(degs, xw):
    def body(degs_ref, xw_ref, oisq_ref, iisq_ref, y1_ref):
        dout = degs_ref[0, :, 0:1]
        din = degs_ref[1, :, 0:1]
        oisq = lax.rsqrt(jnp.maximum(dout, 1.0))
        iisq = lax.rsqrt(jnp.maximum(din, 1.0))
        oisq_ref[...] = oisq
        iisq_ref[...] = iisq
        y1 = xw_ref[...] * oisq
        y1_ref[0, :, 0:32] = y1[:, :32]
        y1_ref[1, :, 0:32] = y1[:, 32:]

    return pl.pallas_call(
        body,
        grid=(NR,),
        in_specs=[
            pl.BlockSpec((2, BR, 128), lambda i: (0, i, 0)),
            pl.BlockSpec((BR, 64), lambda i: (i, 0)),
        ],
        out_specs=(
            pl.BlockSpec((BR, 1), lambda i: (i, 0)),
            pl.BlockSpec((BR, 1), lambda i: (i, 0)),
            pl.BlockSpec((2, BR, 128), lambda i: (0, i, 0)),
        ),
        out_shape=(
            jax.ShapeDtypeStruct((NP, 1), jnp.float32),
            jax.ShapeDtypeStruct((NP, 1), jnp.float32),
            jax.ShapeDtypeStruct((2, NP, 128), jnp.float32),
        ),
    )(degs, xw)


def _stage_b(parts, iisq, oisq, b_enc, w_emb):
    def body(p_ref, ii_ref, oi_ref, b_ref, w_ref, y2_ref):
        agg = jnp.concatenate([p_ref[0, :, 0:32], p_ref[1, :, 0:32]], axis=1)
        h = jnp.maximum(agg * ii_ref[...] + b_ref[...], 0.0)
        y2 = jnp.dot(h * oi_ref[...], w_ref[...],
                     preferred_element_type=jnp.float32)
        y2_ref[0, :, 0:16] = y2[:, :16]
        y2_ref[1, :, 0:16] = y2[:, 16:]

    return pl.pallas_call(
        body,
        grid=(NR,),
        in_specs=[
            pl.BlockSpec((2, BR, 128), lambda i: (0, i, 0)),
            pl.BlockSpec((BR, 1), lambda i: (i, 0)),
            pl.BlockSpec((BR, 1), lambda i: (i, 0)),
            pl.BlockSpec((1, 64), lambda i: (0, 0)),
            pl.BlockSpec((64, 32), lambda i: (0, 0)),
        ],
        out_specs=pl.BlockSpec((2, BR, 128), lambda i: (0, i, 0)),
        out_shape=jax.ShapeDtypeStruct((2, NP, 128), jnp.float32),
    )(parts, iisq, oisq, b_enc, w_emb)


def _stage_c(parts, iisq, oisq, b_emb):
    def body(p_ref, ii_ref, oi_ref, b_ref, z_ref, y3_ref):
        z = jnp.concatenate([p_ref[0, :, 0:16], p_ref[1, :, 0:16]], axis=1) \
            * ii_ref[...] + b_ref[...]
        z_ref[...] = z
        y3 = z * oi_ref[...]
        y3_ref[0, :, 0:16] = y3[:, :16]
        y3_ref[1, :, 0:16] = y3[:, 16:]

    return pl.pallas_call(
        body,
        grid=(NR,),
        in_specs=[
            pl.BlockSpec((2, BR, 128), lambda i: (0, i, 0)),
            pl.BlockSpec((BR, 1), lambda i: (i, 0)),
            pl.BlockSpec((BR, 1), lambda i: (i, 0)),
            pl.BlockSpec((1, 32), lambda i: (0, 0)),
        ],
        out_specs=(
            pl.BlockSpec((BR, 32), lambda i: (i, 0)),
            pl.BlockSpec((2, BR, 128), lambda i: (0, i, 0)),
        ),
        out_shape=(
            jax.ShapeDtypeStruct((NP, 32), jnp.float32),
            jax.ShapeDtypeStruct((2, NP, 128), jnp.float32),
        ),
    )(parts, iisq, oisq, b_emb)


def _stage_d(parts, iisq, b_dec, w_dec):
    def body(p_ref, ii_ref, b_ref, w_ref, rec_ref):
        agg = jnp.concatenate([p_ref[0, :, 0:16], p_ref[1, :, 0:16]],
                              axis=1) * ii_ref[...]
        rec_ref[...] = jnp.dot(agg, w_ref[...],
                               preferred_element_type=jnp.float32) + b_ref[...]

    return pl.pallas_call(
        body,
        grid=(NR,),
        in_specs=[
            pl.BlockSpec((2, BR, 128), lambda i: (0, i, 0)),
            pl.BlockSpec((BR, 1), lambda i: (i, 0)),
            pl.BlockSpec((1, 128), lambda i: (0, 0)),
            pl.BlockSpec((32, 128), lambda i: (0, 0)),
        ],
        out_specs=pl.BlockSpec((BR, 128), lambda i: (i, 0)),
        out_shape=jax.ShapeDtypeStruct((NP, 128), jnp.float32),
    )(parts, iisq, b_dec, w_dec)


def kernel(features, edge_index, W_enc, b_enc, W_emb, b_emb, W_dec, b_dec):
    idx3 = jnp.pad(edge_index.reshape(2, EE // CH, CH),
                   ((0, 0), (0, (EP - EE) // CH), (0, 0)),
                   constant_values=NN)

    feats = jnp.pad(features, ((0, NP - NN), (0, 0)))
    zeros16 = jnp.zeros((NP, 16), jnp.float32)
    ones16 = jnp.ones((CH, 16), jnp.float32)
    zeros32 = jnp.zeros((NP, 32), jnp.float32)

    xw = _matmul_xw(feats, W_enc)
    degs = _degree_kernel(idx3, zeros16, ones16)
    oisq, iisq, y1 = _stage_a(degs, xw)
    agg1 = _propagate(y1, idx3, zeros32, 32)
    y2 = _stage_b(agg1, iisq, oisq, b_enc.reshape(1, -1), W_emb)
    agg2 = _propagate(y2, idx3, zeros16, 16)
    z_full, y3 = _stage_c(agg2, iisq, oisq, b_emb.reshape(1, -1))
    agg3 = _propagate(y3, idx3, zeros16, 16)
    recon_full = _stage_d(agg3, iisq, b_dec.reshape(1, -1), W_dec)

    return (z_full[:NN], recon_full[:NN])
